# baseline (speedup 1.0000x reference)
import math

import jax
import jax.numpy as jnp
import numpy as np
from jax import lax
from jax.experimental import pallas as pl
from jax.experimental.pallas import tpu as pltpu
from jax.experimental.pallas import tpu_sc as plsc

D_MODEL = 768
_SCALE = math.sqrt(float(D_MODEL))
_LANES = 16


def _pos_encoding(seq_len: int, d_model: int) -> np.ndarray:
    pos = np.arange(seq_len, dtype=np.float32)[:, None]
    i = np.arange(d_model, dtype=np.float32)[None, :]
    angle_rates = 1.0 / np.power(10000.0, (2.0 * np.floor(i / 2.0)) / d_model)
    angles = pos * angle_rates
    pe = np.zeros((seq_len, d_model), dtype=np.float32)
    pe[:, 0::2] = np.sin(angles[:, 0::2])
    pe[:, 1::2] = np.cos(angles[:, 1::2])
    return pe


def _make_sc_call(B: int, T: int, V: int, D: int):
    info = plsc.get_sparse_core_info()
    NC, NS = info.num_cores, info.num_subcores
    NW = NC * NS
    assert T % NW == 0
    t_per_w = T // NW
    n_vec = (t_per_w * D) // _LANES

    mesh = plsc.VectorSubcoreMesh(core_axis_name="c", subcore_axis_name="s")

    @jax.jit
    def call(idx_w, table, pe):
        @pl.kernel(
            mesh=mesh,
            out_type=jax.ShapeDtypeStruct((B * T, D), jnp.float32),
            scratch_types=[
                pltpu.VMEM((B, t_per_w), jnp.int32),
                pltpu.VMEM((t_per_w, D), jnp.float32),
                pltpu.VMEM((t_per_w, D), jnp.float32),
                pltpu.SemaphoreType.DMA,
            ],
        )
        def k(idx_hbm, table_hbm, pe_hbm, out_hbm, idx_v, pe_v, g_v, sem):
            wid = lax.axis_index("s") * NC + lax.axis_index("c")
            t0 = wid * t_per_w
            pltpu.sync_copy(idx_hbm.at[wid], idx_v)
            pltpu.sync_copy(pe_hbm.at[pl.ds(t0, t_per_w)], pe_v)
            for b in range(B):
                pltpu.async_copy(table_hbm.at[idx_v.at[b]], g_v, sem).wait()

                def body(r, _):
                    for j in range(D // _LANES):
                        sl = pl.ds(j * _LANES, _LANES)
                        g_v[r, sl] = g_v[r, sl] * _SCALE + pe_v[r, sl]
                    return _

                lax.fori_loop(0, t_per_w, body, None)
                pltpu.sync_copy(g_v, out_hbm.at[pl.ds(b * T + t0, t_per_w)])

        return k(idx_w, table, pe)

    return call


def kernel(token_ids, embedding_table):
    B, T = token_ids.shape
    V, D = embedding_table.shape
    info = plsc.get_sparse_core_info()
    NW = info.num_cores * info.num_subcores
    t_per_w = T // NW
    idx_w = token_ids.reshape(B, NW, t_per_w).transpose(1, 0, 2)
    pe = jnp.asarray(_pos_encoding(T, D))
    call = _make_sc_call(B, T, V, D)
    out = call(idx_w, embedding_table, pe)
    return out.reshape(B, T, D)
